# baseline (speedup 1.0000x reference)
import functools

import jax
import jax.numpy as jnp
from jax import lax
from jax.experimental import pallas as pl
from jax.experimental.pallas import tpu as pltpu
from jax.experimental.pallas import tpu_sc as plsc

N = 10000
D = 128
H = 128
C = 10
E = 320000

NA = 10112
NW = 32
CH = 128
NCH = 80
EPAD = NW * NCH * CH
RPS = NA // 16

_mesh = plsc.VectorSubcoreMesh(core_axis_name="c", subcore_axis_name="s")


@functools.partial(
    pl.kernel,
    out_type=(
        jax.ShapeDtypeStruct((2 * 10240,), jnp.float32),
        jax.ShapeDtypeStruct((2 * 10240,), jnp.float32),
    ),
    mesh=_mesh,
    scratch_types=[
        pltpu.VMEM((NCH, CH), jnp.int32),
        pltpu.VMEM((NCH, CH), jnp.int32),
        pltpu.VMEM((CH,), jnp.float32),
        pltpu.VMEM((640,), jnp.float32),
        pltpu.VMEM_SHARED((10240,), jnp.float32),
        pltpu.VMEM_SHARED((10240,), jnp.float32),
        pltpu.SemaphoreType.DMA,
    ],
)
def _sc_deg(src_hbm, dst_hbm, os_hbm, od_hbm,
            idx_s, idx_d, ones_v, zv, dssp, ddsp, sem):
    c = lax.axis_index("c")
    s = lax.axis_index("s")
    wid = s * 2 + c
    pltpu.sync_copy(src_hbm.at[wid], idx_s)
    pltpu.sync_copy(dst_hbm.at[wid], idx_d)

    def fill_ones(r, _):
        ones_v[pl.ds(r * 16, 16)] = jnp.ones((16,), jnp.float32)
        return 0

    lax.fori_loop(0, CH // 16, fill_ones, 0)

    def fill_zero(r, _):
        zv[pl.ds(r * 16, 16)] = jnp.zeros((16,), jnp.float32)
        return 0

    lax.fori_loop(0, 40, fill_zero, 0)
    zbase = s * 640
    pltpu.sync_copy(zv, dssp.at[pl.ds(zbase, 640)])
    pltpu.sync_copy(zv, ddsp.at[pl.ds(zbase, 640)])
    plsc.subcore_barrier()

    def grp_s(g, _):
        for k in range(4):
            pltpu.async_copy(ones_v, dssp.at[idx_s.at[g * 4 + k]], sem, add=True)
        for k in range(4):
            pltpu.make_async_copy(ones_v, dssp.at[idx_s.at[g * 4 + k]], sem).wait()
        return 0

    lax.fori_loop(0, NCH // 4, grp_s, 0)

    def grp_d(g, _):
        for k in range(4):
            pltpu.async_copy(ones_v, ddsp.at[idx_d.at[g * 4 + k]], sem, add=True)
        for k in range(4):
            pltpu.make_async_copy(ones_v, ddsp.at[idx_d.at[g * 4 + k]], sem).wait()
        return 0

    lax.fori_loop(0, NCH // 4, grp_d, 0)
    plsc.subcore_barrier()
    obase = c * 10240 + zbase
    pltpu.sync_copy(dssp.at[pl.ds(zbase, 640)], os_hbm.at[pl.ds(obase, 640)])
    pltpu.sync_copy(ddsp.at[pl.ds(zbase, 640)], od_hbm.at[pl.ds(obase, 640)])


@functools.partial(
    pl.kernel,
    out_type=jax.ShapeDtypeStruct((2, NA, D), jnp.float32),
    mesh=_mesh,
    scratch_types=[
        pltpu.VMEM((4, CH), jnp.int32),
        pltpu.VMEM((4, CH), jnp.int32),
        pltpu.VMEM((CH, D), jnp.float32),
        pltpu.VMEM((CH, D), jnp.float32),
        pltpu.VMEM((CH, D), jnp.float32),
        pltpu.VMEM_SHARED((NA, D), jnp.float32),
        pltpu.SemaphoreType.DMA,
        pltpu.SemaphoreType.DMA,
        pltpu.SemaphoreType.DMA,
    ],
)
def _sc_agg(xn_hbm, src_hbm, dst_hbm, out_hbm,
            isr, idr, rows0, rows1, rows2, aggsp, isem, gsem, ssem):
    c = lax.axis_index("c")
    s = lax.axis_index("s")
    wid = s * 2 + c
    rows = (rows0, rows1, rows2)

    def fire_idx(j):
        jw = lax.rem(jnp.int32(j), jnp.int32(NCH))
        pltpu.async_copy(src_hbm.at[wid, jw], isr.at[j % 4], isem)
        pltpu.async_copy(dst_hbm.at[wid, jw], idr.at[j % 4], isem)

    def wait_idx(j):
        jw = lax.rem(jnp.int32(j), jnp.int32(NCH))
        pltpu.make_async_copy(src_hbm.at[wid, jw], isr.at[j % 4], isem).wait()
        pltpu.make_async_copy(dst_hbm.at[wid, jw], idr.at[j % 4], isem).wait()

    def fire_gather(j, b):
        pltpu.async_copy(xn_hbm.at[isr.at[j % 4]], rows[b], gsem)

    def wait_gather(j, b):
        pltpu.make_async_copy(xn_hbm.at[isr.at[j % 4]], rows[b], gsem).wait()

    def fire_scat(j, b):
        pltpu.async_copy(rows[b], aggsp.at[idr.at[j % 4]], ssem, add=True)

    def wait_scat(j, b):
        pltpu.make_async_copy(rows[b], aggsp.at[idr.at[j % 4]], ssem).wait()

    fire_idx(0)
    fire_idx(1)
    fire_idx(2)

    def zrow(r, _):
        for k in range(D // 16):
            rows0[r, pl.ds(k * 16, 16)] = jnp.zeros((16,), jnp.float32)
        return 0

    lax.fori_loop(0, CH, zrow, 0)
    base = s * RPS

    def zcp(k, _):
        pltpu.sync_copy(rows0, aggsp.at[pl.ds(base + k * CH, CH)])
        return 0

    lax.fori_loop(0, RPS // CH, zcp, 0)
    pltpu.sync_copy(rows0, aggsp.at[pl.ds(base + RPS - CH, CH)])
    plsc.subcore_barrier()

    wait_idx(0)
    fire_gather(0, 0)

    wait_gather(0, 0)
    fire_scat(0, 0)
    wait_idx(1)
    fire_gather(1, 1)
    fire_idx(3)

    wait_gather(1, 1)
    fire_scat(1, 1)
    wait_idx(2)
    fire_gather(2, 2)

    def body(g, _):
        for t in range(3):
            j = 2 + g * 3 + t
            b = (2 + t) % 3
            wait_gather(j, b)
            fire_scat(j, b)
            wait_idx(j + 1)
            wait_scat(j - 2, (b + 1) % 3)
            fire_gather(j + 1, (b + 1) % 3)
            fire_idx(j + 2)
        return 0

    lax.fori_loop(0, (NCH - 2) // 3, body, 0)
    wait_gather(NCH, NCH % 3)
    wait_idx(NCH + 1)
    wait_scat(NCH - 2, (NCH - 2) % 3)
    wait_scat(NCH - 1, (NCH - 1) % 3)
    plsc.subcore_barrier()
    pltpu.sync_copy(aggsp.at[pl.ds(base, RPS)],
                    out_hbm.at[c, pl.ds(base, RPS)])


BN = NA // 16


def _prep_body(ds_ref, x_ref, o_ref):
    i = pl.program_id(0)
    ds = ds_ref[...]
    norm = lax.rsqrt(jnp.maximum(ds[:, 0:1] + ds[:, 1:2], 1.0))
    rowid = i * BN + lax.broadcasted_iota(jnp.int32, (BN, 1), 0)
    o_ref[...] = jnp.where(rowid < N, x_ref[...] * norm, 0.0)


def _prep(dsT, x):
    return pl.pallas_call(
        _prep_body,
        grid=(NA // BN,),
        in_specs=[
            pl.BlockSpec((BN, 2), lambda i: (i, 0)),
            pl.BlockSpec((BN, D), lambda i: (i, 0)),
        ],
        out_specs=pl.BlockSpec((BN, D), lambda i: (i, 0)),
        out_shape=jax.ShapeDtypeStruct((NA, D), jnp.float32),
    )(dsT, x)


def _l1_body(agg_ref, dd_ref, ds_ref, w_ref, b_ref, o_ref):
    a = agg_ref[0] + agg_ref[1]
    dd = dd_ref[...]
    ds = ds_ref[...]
    nd = lax.rsqrt(jnp.maximum(dd[:, 0:1] + dd[:, 1:2], 1.0))
    ns = lax.rsqrt(jnp.maximum(ds[:, 0:1] + ds[:, 1:2], 1.0))
    h = jnp.dot(a * nd, w_ref[...], preferred_element_type=jnp.float32)
    o_ref[...] = jnp.maximum(h + b_ref[...], 0.0) * ns


def _l1(agg, ddT, dsT, W1, b1):
    return pl.pallas_call(
        _l1_body,
        grid=(NA // BN,),
        in_specs=[
            pl.BlockSpec((2, BN, D), lambda i: (0, i, 0)),
            pl.BlockSpec((BN, 2), lambda i: (i, 0)),
            pl.BlockSpec((BN, 2), lambda i: (i, 0)),
            pl.BlockSpec((D, H), lambda i: (0, 0)),
            pl.BlockSpec((1, H), lambda i: (0, 0)),
        ],
        out_specs=pl.BlockSpec((BN, H), lambda i: (i, 0)),
        out_shape=jax.ShapeDtypeStruct((NA, H), jnp.float32),
    )(agg, ddT, dsT, W1, b1)


def _l2_body(agg_ref, dd_ref, w_ref, b_ref, wl_ref, bl_ref, o_ref, acc_ref):
    i = pl.program_id(0)
    a = agg_ref[0] + agg_ref[1]
    dd = dd_ref[...]
    nd = lax.rsqrt(jnp.maximum(dd[:, 0:1] + dd[:, 1:2], 1.0))
    y = jnp.dot(a * nd, w_ref[...], preferred_element_type=jnp.float32)
    y = y + b_ref[...]
    rows = i * BN + lax.broadcasted_iota(jnp.int32, (BN, 1), 0)
    y = jnp.where(rows < N, y, -jnp.inf)
    bm = jnp.max(y, axis=0, keepdims=True)

    @pl.when(i == 0)
    def _():
        acc_ref[...] = bm

    @pl.when(i > 0)
    def _():
        acc_ref[...] = jnp.maximum(acc_ref[...], bm)

    @pl.when(i == pl.num_programs(0) - 1)
    def _():
        pooled = jnp.maximum(acc_ref[...], 0.0)
        o_ref[...] = (
            jnp.dot(pooled, wl_ref[...], preferred_element_type=jnp.float32)
            + bl_ref[...]
        )


def _l2(agg, ddT, W2, b2, wlp, blp):
    return pl.pallas_call(
        _l2_body,
        grid=(NA // BN,),
        in_specs=[
            pl.BlockSpec((2, BN, D), lambda i: (0, i, 0)),
            pl.BlockSpec((BN, 2), lambda i: (i, 0)),
            pl.BlockSpec((D, H), lambda i: (0, 0)),
            pl.BlockSpec((1, H), lambda i: (0, 0)),
            pl.BlockSpec((H, 128), lambda i: (0, 0)),
            pl.BlockSpec((1, 128), lambda i: (0, 0)),
        ],
        out_specs=pl.BlockSpec((1, 128), lambda i: (0, 0)),
        out_shape=jax.ShapeDtypeStruct((1, 128), jnp.float32),
        scratch_shapes=[pltpu.VMEM((1, H), jnp.float32)],
    )(agg, ddT, W2, b2, wlp, blp)


def kernel(x, edge_index, W1, b1, W2, b2, Wl, bl):
    f32 = jnp.float32
    src = edge_index[0]
    dst = edge_index[1]
    npad = EPAD - E
    ar = jnp.arange(npad, dtype=jnp.int32)
    srcp = jnp.concatenate([src, N + 32 + (ar % 32)])
    dstp = jnp.concatenate([dst, N + (ar % 32)])
    src3 = srcp.reshape(NW, NCH, CH)
    dst3 = dstp.reshape(NW, NCH, CH)

    degs, degd = _sc_deg(src3, dst3)
    dsT = jnp.transpose(degs.reshape(2, 10240)[:, :NA])
    ddT = jnp.transpose(degd.reshape(2, 10240)[:, :NA])

    xn = _prep(dsT, x)
    agg1 = _sc_agg(xn, src3, dst3)
    h1n = _l1(agg1, ddT, dsT, W1, b1.reshape(1, H))
    agg2 = _sc_agg(h1n, src3, dst3)

    wlp = jnp.zeros((H, 128), f32).at[:, :C].set(Wl)
    blp = jnp.zeros((1, 128), f32).at[:, :C].set(bl)
    res = _l2(agg2, ddT, W2, b2.reshape(1, H), wlp, blp)
    return res[:, :C]


# speedup vs baseline: 1.0245x; 1.0245x over previous
import functools

import jax
import jax.numpy as jnp
from jax import lax
from jax.experimental import pallas as pl
from jax.experimental.pallas import tpu as pltpu
from jax.experimental.pallas import tpu_sc as plsc

N = 10000
D = 128
H = 128
C = 10
E = 320000

NP = 10240
NW = 32
CH = 128
NCH = 80
EPAD = NW * NCH * CH
ROWS_PER_SUB = NP // 16

_mesh = plsc.VectorSubcoreMesh(core_axis_name="c", subcore_axis_name="s")


@functools.partial(
    pl.kernel,
    out_type=(
        jax.ShapeDtypeStruct((2, NP), jnp.float32),
        jax.ShapeDtypeStruct((2, NP), jnp.float32),
    ),
    mesh=_mesh,
    scratch_types=[
        pltpu.VMEM((NCH, CH), jnp.int32),
        pltpu.VMEM((NCH, CH), jnp.int32),
        pltpu.VMEM((CH,), jnp.float32),
        pltpu.VMEM((ROWS_PER_SUB,), jnp.float32),
        pltpu.VMEM_SHARED((NP,), jnp.float32),
        pltpu.VMEM_SHARED((NP,), jnp.float32),
        pltpu.SemaphoreType.DMA,
    ],
)
def _sc_deg(src_hbm, dst_hbm, os_hbm, od_hbm,
            idx_s, idx_d, ones_v, zv, dssp, ddsp, sem):
    c = lax.axis_index("c")
    s = lax.axis_index("s")
    wid = s * 2 + c
    pltpu.sync_copy(src_hbm.at[wid], idx_s)
    pltpu.sync_copy(dst_hbm.at[wid], idx_d)

    def fill_ones(r, _):
        ones_v[pl.ds(r * 16, 16)] = jnp.ones((16,), jnp.float32)
        return 0

    lax.fori_loop(0, CH // 16, fill_ones, 0)

    def fill_zero(r, _):
        zv[pl.ds(r * 16, 16)] = jnp.zeros((16,), jnp.float32)
        return 0

    lax.fori_loop(0, ROWS_PER_SUB // 16, fill_zero, 0)
    base = s * ROWS_PER_SUB
    pltpu.sync_copy(zv, dssp.at[pl.ds(base, ROWS_PER_SUB)])
    pltpu.sync_copy(zv, ddsp.at[pl.ds(base, ROWS_PER_SUB)])
    plsc.subcore_barrier()

    def grp_s(g, _):
        for k in range(4):
            pltpu.async_copy(ones_v, dssp.at[idx_s.at[g * 4 + k]], sem, add=True)
        for k in range(4):
            pltpu.make_async_copy(ones_v, dssp.at[idx_s.at[g * 4 + k]], sem).wait()
        return 0

    lax.fori_loop(0, NCH // 4, grp_s, 0)

    def grp_d(g, _):
        for k in range(4):
            pltpu.async_copy(ones_v, ddsp.at[idx_d.at[g * 4 + k]], sem, add=True)
        for k in range(4):
            pltpu.make_async_copy(ones_v, ddsp.at[idx_d.at[g * 4 + k]], sem).wait()
        return 0

    lax.fori_loop(0, NCH // 4, grp_d, 0)
    plsc.subcore_barrier()
    pltpu.sync_copy(dssp.at[pl.ds(base, ROWS_PER_SUB)],
                    os_hbm.at[c, pl.ds(base, ROWS_PER_SUB)])
    pltpu.sync_copy(ddsp.at[pl.ds(base, ROWS_PER_SUB)],
                    od_hbm.at[c, pl.ds(base, ROWS_PER_SUB)])


@functools.partial(
    pl.kernel,
    out_type=jax.ShapeDtypeStruct((2, NP, D), jnp.float32),
    mesh=_mesh,
    scratch_types=[
        pltpu.VMEM((CH,), jnp.int32),
        pltpu.VMEM((CH,), jnp.int32),
        pltpu.VMEM((CH,), jnp.int32),
        pltpu.VMEM((CH,), jnp.int32),
        pltpu.VMEM((CH, D), jnp.float32),
        pltpu.VMEM((CH, D), jnp.float32),
        pltpu.VMEM_SHARED((NP, D), jnp.float32),
        pltpu.SemaphoreType.DMA,
        pltpu.SemaphoreType.DMA,
    ],
)
def _sc_agg(xn_hbm, src_hbm, dst_hbm, out_hbm,
            is0, is1, id0, id1, rows0, rows1, aggsp, isem, gsem):
    c = lax.axis_index("c")
    s = lax.axis_index("s")
    wid = s * 2 + c
    isb = (is0, is1)
    idb = (id0, id1)
    rows = (rows0, rows1)

    def fire_idx(j, b):
        jw = lax.rem(jnp.int32(j), jnp.int32(NCH))
        pltpu.async_copy(src_hbm.at[wid, jw], isb[b], isem)
        pltpu.async_copy(dst_hbm.at[wid, jw], idb[b], isem)

    def wait_idx(j, b):
        jw = lax.rem(jnp.int32(j), jnp.int32(NCH))
        pltpu.make_async_copy(src_hbm.at[wid, jw], isb[b], isem).wait()
        pltpu.make_async_copy(dst_hbm.at[wid, jw], idb[b], isem).wait()

    fire_idx(0, 0)
    fire_idx(1, 1)

    def zrow(r, _):
        for k in range(D // 16):
            rows0[r, pl.ds(k * 16, 16)] = jnp.zeros((16,), jnp.float32)
        return 0

    lax.fori_loop(0, CH, zrow, 0)
    base = s * ROWS_PER_SUB

    def zcp(k, _):
        pltpu.sync_copy(rows0, aggsp.at[pl.ds(base + k * CH, CH)])
        return 0

    lax.fori_loop(0, ROWS_PER_SUB // CH, zcp, 0)
    plsc.subcore_barrier()

    wait_idx(0, 0)
    pltpu.async_copy(xn_hbm.at[is0], rows0, gsem)

    def grp(g, _):
        for b in range(2):
            j = g * 2 + b
            pltpu.make_async_copy(xn_hbm.at[isb[b]], rows[b], gsem).wait()
            wait_idx(j + 1, 1 - b)
            pltpu.async_copy(xn_hbm.at[isb[1 - b]], rows[1 - b], gsem)
            pltpu.sync_copy(rows[b], aggsp.at[idb[b]], add=True)
            fire_idx(j + 2, b)
        return 0

    lax.fori_loop(0, NCH // 2, grp, 0)
    pltpu.make_async_copy(xn_hbm.at[is0], rows0, gsem).wait()
    wait_idx(NCH + 1, 1)
    plsc.subcore_barrier()
    pltpu.sync_copy(aggsp.at[pl.ds(base, ROWS_PER_SUB)],
                    out_hbm.at[c, pl.ds(base, ROWS_PER_SUB)])


BN = 1024


def _prep_body(ds_ref, x_ref, o_ref):
    ds = ds_ref[...]
    norm = lax.rsqrt(jnp.maximum(ds[:, 0:1] + ds[:, 1:2], 1.0))
    o_ref[...] = x_ref[...] * norm


def _prep(dsT, xp):
    return pl.pallas_call(
        _prep_body,
        grid=(NP // BN,),
        in_specs=[
            pl.BlockSpec((BN, 2), lambda i: (i, 0)),
            pl.BlockSpec((BN, D), lambda i: (i, 0)),
        ],
        out_specs=pl.BlockSpec((BN, D), lambda i: (i, 0)),
        out_shape=jax.ShapeDtypeStruct((NP, D), jnp.float32),
    )(dsT, xp)


def _l1_body(agg_ref, dd_ref, ds_ref, w_ref, b_ref, o_ref):
    a = agg_ref[0] + agg_ref[1]
    dd = dd_ref[...]
    ds = ds_ref[...]
    nd = lax.rsqrt(jnp.maximum(dd[:, 0:1] + dd[:, 1:2], 1.0))
    ns = lax.rsqrt(jnp.maximum(ds[:, 0:1] + ds[:, 1:2], 1.0))
    h = jnp.dot(a * nd, w_ref[...], preferred_element_type=jnp.float32)
    o_ref[...] = jnp.maximum(h + b_ref[...], 0.0) * ns


def _l1(agg, ddT, dsT, W1, b1):
    return pl.pallas_call(
        _l1_body,
        grid=(NP // BN,),
        in_specs=[
            pl.BlockSpec((2, BN, D), lambda i: (0, i, 0)),
            pl.BlockSpec((BN, 2), lambda i: (i, 0)),
            pl.BlockSpec((BN, 2), lambda i: (i, 0)),
            pl.BlockSpec((D, H), lambda i: (0, 0)),
            pl.BlockSpec((1, H), lambda i: (0, 0)),
        ],
        out_specs=pl.BlockSpec((BN, H), lambda i: (i, 0)),
        out_shape=jax.ShapeDtypeStruct((NP, H), jnp.float32),
    )(agg, ddT, dsT, W1, b1)


def _l2_body(agg_ref, dd_ref, w_ref, b_ref, wl_ref, bl_ref, o_ref, acc_ref):
    i = pl.program_id(0)
    a = agg_ref[0] + agg_ref[1]
    dd = dd_ref[...]
    nd = lax.rsqrt(jnp.maximum(dd[:, 0:1] + dd[:, 1:2], 1.0))
    y = jnp.dot(a * nd, w_ref[...], preferred_element_type=jnp.float32)
    y = y + b_ref[...]
    rows = i * BN + lax.broadcasted_iota(jnp.int32, (BN, 1), 0)
    y = jnp.where(rows < N, y, -jnp.inf)
    bm = jnp.max(y, axis=0, keepdims=True)

    @pl.when(i == 0)
    def _():
        acc_ref[...] = bm

    @pl.when(i > 0)
    def _():
        acc_ref[...] = jnp.maximum(acc_ref[...], bm)

    @pl.when(i == pl.num_programs(0) - 1)
    def _():
        pooled = jnp.maximum(acc_ref[...], 0.0)
        o_ref[...] = (
            jnp.dot(pooled, wl_ref[...], preferred_element_type=jnp.float32)
            + bl_ref[...]
        )


def _l2(agg, ddT, W2, b2, wlp, blp):
    return pl.pallas_call(
        _l2_body,
        grid=(NP // BN,),
        in_specs=[
            pl.BlockSpec((2, BN, D), lambda i: (0, i, 0)),
            pl.BlockSpec((BN, 2), lambda i: (i, 0)),
            pl.BlockSpec((D, H), lambda i: (0, 0)),
            pl.BlockSpec((1, H), lambda i: (0, 0)),
            pl.BlockSpec((H, 128), lambda i: (0, 0)),
            pl.BlockSpec((1, 128), lambda i: (0, 0)),
        ],
        out_specs=pl.BlockSpec((1, 128), lambda i: (0, 0)),
        out_shape=jax.ShapeDtypeStruct((1, 128), jnp.float32),
        scratch_shapes=[pltpu.VMEM((1, H), jnp.float32)],
    )(agg, ddT, W2, b2, wlp, blp)


def kernel(x, edge_index, W1, b1, W2, b2, Wl, bl):
    f32 = jnp.float32
    xp = jnp.zeros((NP, D), f32).at[:N].set(x)

    src = edge_index[0]
    dst = edge_index[1]
    npad = EPAD - E
    ar = jnp.arange(npad, dtype=jnp.int32)
    srcp = jnp.concatenate([src, N + 32 + (ar % 32)])
    dstp = jnp.concatenate([dst, N + (ar % 32)])
    src3 = srcp.reshape(NW, NCH, CH)
    dst3 = dstp.reshape(NW, NCH, CH)

    degs, degd = _sc_deg(src3, dst3)
    dsT = jnp.transpose(degs)
    ddT = jnp.transpose(degd)

    xn = _prep(dsT, xp)
    agg1 = _sc_agg(xn, src3, dst3)
    h1n = _l1(agg1, ddT, dsT, W1, b1.reshape(1, H))
    agg2 = _sc_agg(h1n, src3, dst3)

    wlp = jnp.zeros((H, 128), f32).at[:, :C].set(Wl)
    blp = jnp.zeros((1, 128), f32).at[:, :C].set(bl)
    res = _l2(agg2, ddT, W2, b2.reshape(1, H), wlp, blp)
    return res[:, :C]


# speedup vs baseline: 1.0292x; 1.0045x over previous
import functools

import jax
import jax.numpy as jnp
from jax import lax
from jax.experimental import pallas as pl
from jax.experimental.pallas import tpu as pltpu
from jax.experimental.pallas import tpu_sc as plsc

N = 10000
D = 128
H = 128
C = 10
E = 320000

NP = 10240
NW = 32
CH = 128
NCH = 80
EPAD = NW * NCH * CH
ROWS_PER_SUB = NP // 16

_mesh = plsc.VectorSubcoreMesh(core_axis_name="c", subcore_axis_name="s")


@functools.partial(
    pl.kernel,
    out_type=(
        jax.ShapeDtypeStruct((2, NP), jnp.float32),
        jax.ShapeDtypeStruct((2, NP), jnp.float32),
    ),
    mesh=_mesh,
    scratch_types=[
        pltpu.VMEM((NCH, CH), jnp.int32),
        pltpu.VMEM((NCH, CH), jnp.int32),
        pltpu.VMEM((CH,), jnp.float32),
        pltpu.VMEM((ROWS_PER_SUB,), jnp.float32),
        pltpu.VMEM_SHARED((NP,), jnp.float32),
        pltpu.VMEM_SHARED((NP,), jnp.float32),
        pltpu.SemaphoreType.DMA,
    ],
)
def _sc_deg(src_hbm, dst_hbm, os_hbm, od_hbm,
            idx_s, idx_d, ones_v, zv, dssp, ddsp, sem):
    c = lax.axis_index("c")
    s = lax.axis_index("s")
    wid = s * 2 + c
    pltpu.sync_copy(src_hbm.at[wid], idx_s)
    pltpu.sync_copy(dst_hbm.at[wid], idx_d)

    def fill_ones(r, _):
        ones_v[pl.ds(r * 16, 16)] = jnp.ones((16,), jnp.float32)
        return 0

    lax.fori_loop(0, CH // 16, fill_ones, 0)

    def fill_zero(r, _):
        zv[pl.ds(r * 16, 16)] = jnp.zeros((16,), jnp.float32)
        return 0

    lax.fori_loop(0, ROWS_PER_SUB // 16, fill_zero, 0)
    base = s * ROWS_PER_SUB
    pltpu.sync_copy(zv, dssp.at[pl.ds(base, ROWS_PER_SUB)])
    pltpu.sync_copy(zv, ddsp.at[pl.ds(base, ROWS_PER_SUB)])
    plsc.subcore_barrier()

    def grp_s(g, _):
        for k in range(4):
            pltpu.async_copy(ones_v, dssp.at[idx_s.at[g * 4 + k]], sem, add=True)
        for k in range(4):
            pltpu.make_async_copy(ones_v, dssp.at[idx_s.at[g * 4 + k]], sem).wait()
        return 0

    lax.fori_loop(0, NCH // 4, grp_s, 0)

    def grp_d(g, _):
        for k in range(4):
            pltpu.async_copy(ones_v, ddsp.at[idx_d.at[g * 4 + k]], sem, add=True)
        for k in range(4):
            pltpu.make_async_copy(ones_v, ddsp.at[idx_d.at[g * 4 + k]], sem).wait()
        return 0

    lax.fori_loop(0, NCH // 4, grp_d, 0)
    plsc.subcore_barrier()
    pltpu.sync_copy(dssp.at[pl.ds(base, ROWS_PER_SUB)],
                    os_hbm.at[c, pl.ds(base, ROWS_PER_SUB)])
    pltpu.sync_copy(ddsp.at[pl.ds(base, ROWS_PER_SUB)],
                    od_hbm.at[c, pl.ds(base, ROWS_PER_SUB)])


@functools.partial(
    pl.kernel,
    out_type=jax.ShapeDtypeStruct((2, NP, D), jnp.float32),
    mesh=_mesh,
    scratch_types=[
        pltpu.VMEM((CH,), jnp.int32),
        pltpu.VMEM((CH,), jnp.int32),
        pltpu.VMEM((CH,), jnp.int32),
        pltpu.VMEM((CH,), jnp.int32),
        pltpu.VMEM((CH, D), jnp.float32),
        pltpu.VMEM((CH, D), jnp.float32),
        pltpu.VMEM_SHARED((NP, D), jnp.float32),
        pltpu.SemaphoreType.DMA,
        pltpu.SemaphoreType.DMA,
    ],
)
def _sc_agg(xn_hbm, src_hbm, dst_hbm, out_hbm,
            is0, is1, id0, id1, rows0, rows1, aggsp, isem, gsem):
    c = lax.axis_index("c")
    s = lax.axis_index("s")
    wid = s * 2 + c
    isb = (is0, is1)
    idb = (id0, id1)
    rows = (rows0, rows1)

    def fire_idx(j, b):
        jw = lax.rem(jnp.int32(j), jnp.int32(NCH))
        pltpu.async_copy(src_hbm.at[wid, jw], isb[b], isem)
        pltpu.async_copy(dst_hbm.at[wid, jw], idb[b], isem)

    def wait_idx(j, b):
        jw = lax.rem(jnp.int32(j), jnp.int32(NCH))
        pltpu.make_async_copy(src_hbm.at[wid, jw], isb[b], isem).wait()
        pltpu.make_async_copy(dst_hbm.at[wid, jw], idb[b], isem).wait()

    fire_idx(0, 0)
    fire_idx(1, 1)

    def zrow(r, _):
        for k in range(D // 16):
            rows0[r, pl.ds(k * 16, 16)] = jnp.zeros((16,), jnp.float32)
        return 0

    lax.fori_loop(0, CH, zrow, 0)
    base = s * ROWS_PER_SUB

    def zcp(k, _):
        pltpu.sync_copy(rows0, aggsp.at[pl.ds(base + k * CH, CH)])
        return 0

    lax.fori_loop(0, ROWS_PER_SUB // CH, zcp, 0)
    plsc.subcore_barrier()

    wait_idx(0, 0)
    pltpu.async_copy(xn_hbm.at[is0], rows0, gsem)

    def grp(g, _):
        for b in range(2):
            j = g * 2 + b
            pltpu.make_async_copy(xn_hbm.at[isb[b]], rows[b], gsem).wait()
            wait_idx(j + 1, 1 - b)
            pltpu.async_copy(xn_hbm.at[isb[1 - b]], rows[1 - b], gsem)
            pltpu.sync_copy(rows[b], aggsp.at[idb[b]], add=True)
            fire_idx(j + 2, b)
        return 0

    lax.fori_loop(0, NCH // 2, grp, 0)
    pltpu.make_async_copy(xn_hbm.at[is0], rows0, gsem).wait()
    wait_idx(NCH + 1, 1)
    plsc.subcore_barrier()
    pltpu.sync_copy(aggsp.at[pl.ds(base, ROWS_PER_SUB)],
                    out_hbm.at[c, pl.ds(base, ROWS_PER_SUB)])


BN = 1024


def _prep_body(ds_ref, x_ref, o_ref):
    i = pl.program_id(0)
    ds = ds_ref[...]
    norm = lax.rsqrt(jnp.maximum(ds[:, 0:1] + ds[:, 1:2], 1.0))
    rowid = i * BN + lax.broadcasted_iota(jnp.int32, (BN, 1), 0)
    o_ref[...] = jnp.where(rowid < N, x_ref[...] * norm, 0.0)


def _prep(dsT, x):
    return pl.pallas_call(
        _prep_body,
        grid=(NP // BN,),
        in_specs=[
            pl.BlockSpec((BN, 2), lambda i: (i, 0)),
            pl.BlockSpec((BN, D), lambda i: (i, 0)),
        ],
        out_specs=pl.BlockSpec((BN, D), lambda i: (i, 0)),
        out_shape=jax.ShapeDtypeStruct((NP, D), jnp.float32),
    )(dsT, x)


def _l1_body(agg_ref, dd_ref, ds_ref, w_ref, b_ref, o_ref):
    a = agg_ref[0] + agg_ref[1]
    dd = dd_ref[...]
    ds = ds_ref[...]
    nd = lax.rsqrt(jnp.maximum(dd[:, 0:1] + dd[:, 1:2], 1.0))
    ns = lax.rsqrt(jnp.maximum(ds[:, 0:1] + ds[:, 1:2], 1.0))
    h = jnp.dot(a * nd, w_ref[...], preferred_element_type=jnp.float32)
    o_ref[...] = jnp.maximum(h + b_ref[...], 0.0) * ns


def _l1(agg, ddT, dsT, W1, b1):
    return pl.pallas_call(
        _l1_body,
        grid=(NP // BN,),
        in_specs=[
            pl.BlockSpec((2, BN, D), lambda i: (0, i, 0)),
            pl.BlockSpec((BN, 2), lambda i: (i, 0)),
            pl.BlockSpec((BN, 2), lambda i: (i, 0)),
            pl.BlockSpec((D, H), lambda i: (0, 0)),
            pl.BlockSpec((1, H), lambda i: (0, 0)),
        ],
        out_specs=pl.BlockSpec((BN, H), lambda i: (i, 0)),
        out_shape=jax.ShapeDtypeStruct((NP, H), jnp.float32),
    )(agg, ddT, dsT, W1, b1)


def _l2_body(agg_ref, dd_ref, w_ref, b_ref, wl_ref, bl_ref, o_ref, acc_ref):
    i = pl.program_id(0)
    a = agg_ref[0] + agg_ref[1]
    dd = dd_ref[...]
    nd = lax.rsqrt(jnp.maximum(dd[:, 0:1] + dd[:, 1:2], 1.0))
    y = jnp.dot(a * nd, w_ref[...], preferred_element_type=jnp.float32)
    y = y + b_ref[...]
    rows = i * BN + lax.broadcasted_iota(jnp.int32, (BN, 1), 0)
    y = jnp.where(rows < N, y, -jnp.inf)
    bm = jnp.max(y, axis=0, keepdims=True)

    @pl.when(i == 0)
    def _():
        acc_ref[...] = bm

    @pl.when(i > 0)
    def _():
        acc_ref[...] = jnp.maximum(acc_ref[...], bm)

    @pl.when(i == pl.num_programs(0) - 1)
    def _():
        pooled = jnp.maximum(acc_ref[...], 0.0)
        o_ref[...] = (
            jnp.dot(pooled, wl_ref[...], preferred_element_type=jnp.float32)
            + bl_ref[...]
        )


def _l2(agg, ddT, W2, b2, wlp, blp):
    return pl.pallas_call(
        _l2_body,
        grid=(NP // BN,),
        in_specs=[
            pl.BlockSpec((2, BN, D), lambda i: (0, i, 0)),
            pl.BlockSpec((BN, 2), lambda i: (i, 0)),
            pl.BlockSpec((D, H), lambda i: (0, 0)),
            pl.BlockSpec((1, H), lambda i: (0, 0)),
            pl.BlockSpec((H, 128), lambda i: (0, 0)),
            pl.BlockSpec((1, 128), lambda i: (0, 0)),
        ],
        out_specs=pl.BlockSpec((1, 128), lambda i: (0, 0)),
        out_shape=jax.ShapeDtypeStruct((1, 128), jnp.float32),
        scratch_shapes=[pltpu.VMEM((1, H), jnp.float32)],
    )(agg, ddT, W2, b2, wlp, blp)


def kernel(x, edge_index, W1, b1, W2, b2, Wl, bl):
    f32 = jnp.float32

    src = edge_index[0]
    dst = edge_index[1]
    npad = EPAD - E
    ar = jnp.arange(npad, dtype=jnp.int32)
    srcp = jnp.concatenate([src, N + 32 + (ar % 32)])
    dstp = jnp.concatenate([dst, N + (ar % 32)])
    src3 = srcp.reshape(NW, NCH, CH)
    dst3 = dstp.reshape(NW, NCH, CH)

    degs, degd = _sc_deg(src3, dst3)
    dsT = jnp.transpose(degs)
    ddT = jnp.transpose(degd)

    xn = _prep(dsT, x)
    agg1 = _sc_agg(xn, src3, dst3)
    h1n = _l1(agg1, ddT, dsT, W1, b1.reshape(1, H))
    agg2 = _sc_agg(h1n, src3, dst3)

    wlp = jnp.zeros((H, 128), f32).at[:, :C].set(Wl)
    blp = jnp.zeros((1, 128), f32).at[:, :C].set(bl)
    res = _l2(agg2, ddT, W2, b2.reshape(1, H), wlp, blp)
    return res[:, :C]
